# Initial kernel scaffold; baseline (speedup 1.0000x reference)
import jax
import jax.numpy as jnp
from jax.experimental import pallas as pl


def kernel(features, lorentz_vectors, coordinates, mask, W1, b1, W2, b2, Watt, batt):
    raise NotImplementedError("write your pallas kernel here")


# speedup vs baseline: 1.8578x; 1.8578x over previous
import math

import jax
import jax.numpy as jnp
from jax.experimental import pallas as pl
from jax.experimental.pallas import tpu as pltpu

EPS = 1e-8
M = 256
K = 16
PI = math.pi


def _delta_phi(a, b):
    return (a - b + PI) % (2 * PI) - PI


def _fps_body(eta_ref, phi_ref, rnd_ref, out_ref):
    eta = eta_ref[...]
    phi = phi_ref[...]
    rnd = rnd_ref[...]
    P, B = eta.shape
    row = jax.lax.broadcasted_iota(jnp.int32, (P, B), 0)
    mx0 = jnp.max(rnd, axis=0, keepdims=True)
    cur = jnp.min(jnp.where(rnd == mx0, row, P), axis=0, keepdims=True)
    min_d = jnp.full((P, B), jnp.inf, dtype=jnp.float32)

    def step(i, carry):
        min_d, cur = carry
        out_ref[pl.ds(i, 1), :] = cur
        sel = row == cur
        ce = jnp.sum(jnp.where(sel, eta, 0.0), axis=0, keepdims=True)
        cp = jnp.sum(jnp.where(sel, phi, 0.0), axis=0, keepdims=True)
        d = (eta - ce) ** 2 + _delta_phi(phi, cp) ** 2
        min_d = jnp.minimum(min_d, d)
        mx = jnp.max(min_d, axis=0, keepdims=True)
        cur = jnp.min(jnp.where(min_d == mx, row, P), axis=0, keepdims=True)
        return min_d, cur

    jax.lax.fori_loop(0, M, step, (min_d, cur))


def _fps_call(etaT, phiT, rndT):
    P, B = etaT.shape
    return pl.pallas_call(
        _fps_body,
        out_shape=jax.ShapeDtypeStruct((M, B), jnp.int32),
    )(etaT, phiT, rndT)


def _knn_body(coord_ref, cidx_ref, featT_ref, lvT_ref,
              nidx_ref, cF_ref, cLV_ref):
    coord = coord_ref[...]
    P = coord.shape[0]
    etac = coord[:, 0:1]
    phic = coord[:, 1:2]
    cidx = cidx_ref[...]
    iota_p = jax.lax.broadcasted_iota(jnp.int32, (P, M), 0)
    sel = iota_p == cidx
    onehotT = jnp.where(sel, 1.0, 0.0)
    dn = (((0,), (0,)), ((), ()))
    hi = jax.lax.Precision.HIGHEST
    cF_ref[...] = jax.lax.dot_general(onehotT, featT_ref[...], dn,
                                      precision=hi,
                                      preferred_element_type=jnp.float32)
    cLV_ref[...] = jax.lax.dot_general(onehotT, lvT_ref[...], dn,
                                       precision=hi,
                                       preferred_element_type=jnp.float32)
    qeta = jnp.sum(jnp.where(sel, etac, 0.0), axis=0, keepdims=True)
    qphi = jnp.sum(jnp.where(sel, phic, 0.0), axis=0, keepdims=True)
    deta = qeta - etac
    dphi = _delta_phi(qphi, phic)
    dist = deta * deta + dphi * dphi

    def extract(k, dist):
        mn = jnp.min(dist, axis=0, keepdims=True)
        sel = dist == mn
        idx = jnp.min(jnp.where(sel, iota_p, P), axis=0, keepdims=True)
        nidx_ref[pl.ds(k, 1), :] = idx
        return jnp.where(iota_p == idx, jnp.inf, dist)

    jax.lax.fori_loop(0, K, extract, dist)


def _knn_call(coordT8, cidx3, featT, lvT8):
    B, P, C = featT.shape
    return pl.pallas_call(
        _knn_body,
        grid=(B,),
        in_specs=[
            pl.BlockSpec((None, P, 8), lambda b: (b, 0, 0)),
            pl.BlockSpec((None, 1, M), lambda b: (b, 0, 0)),
            pl.BlockSpec((None, P, C), lambda b: (b, 0, 0)),
            pl.BlockSpec((None, P, 8), lambda b: (b, 0, 0)),
        ],
        out_specs=[
            pl.BlockSpec((None, K, M), lambda b: (b, 0, 0)),
            pl.BlockSpec((None, M, C), lambda b: (b, 0, 0)),
            pl.BlockSpec((None, M, 8), lambda b: (b, 0, 0)),
        ],
        out_shape=[
            jax.ShapeDtypeStruct((B, K, M), jnp.int32),
            jax.ShapeDtypeStruct((B, M, C), jnp.float32),
            jax.ShapeDtypeStruct((B, M, 8), jnp.float32),
        ],
    )(coordT8, cidx3, featT, lvT8)


def _mlp_body(nidx_ref, featT_ref, lvT_ref, cF_ref, cLV_ref, noise_ref,
              w1_ref, w2_ref, watt_ref, b1_ref, b2_ref,
              out_ref, lv_ref, msg_s, nlv_s, att_s):
    featT = featT_ref[...]
    lvT = lvT_ref[...]
    P = featT.shape[0]
    nidx = nidx_ref[...]
    cF = cF_ref[...]
    cLV = cLV_ref[...]
    dnT = (((0,), (0,)), ((), ()))
    dnN = (((1,), (0,)), ((), ()))
    hi = jax.lax.Precision.HIGHEST
    f32 = jnp.float32

    pxi, pyi, pzi, ei = cLV[:, 0:1], cLV[:, 1:2], cLV[:, 2:3], cLV[:, 3:4]
    pti = jnp.sqrt(pxi * pxi + pyi * pyi + EPS)
    rapi = 0.5 * jnp.log(jnp.maximum(ei + pzi, EPS) / jnp.maximum(ei - pzi, EPS))
    phii = jnp.arctan2(pyi, pxi)
    iota_p = jax.lax.broadcasted_iota(jnp.int32, (P, M), 0)

    for k in range(K):
        oh = jnp.where(iota_p == nidx[k:k + 1, :], 1.0, 0.0)
        nF = jax.lax.dot_general(oh, featT, dnT, precision=hi,
                                 preferred_element_type=f32)
        nLV = jax.lax.dot_general(oh, lvT, dnT, precision=hi,
                                  preferred_element_type=f32)
        nlv_s[k * M:(k + 1) * M, :] = nLV
        nz = nLV + noise_ref[k]
        pxj, pyj, pzj, ej = nz[:, 0:1], nz[:, 1:2], nz[:, 2:3], nz[:, 3:4]
        ptj = jnp.sqrt(pxj * pxj + pyj * pyj + EPS)
        rapj = 0.5 * jnp.log(jnp.maximum(ej + pzj, EPS) /
                             jnp.maximum(ej - pzj, EPS))
        phij = jnp.arctan2(pyj, pxj)
        delta = jnp.sqrt((rapi - rapj) ** 2 + _delta_phi(phii, phij) ** 2 + EPS)
        lndelta = jnp.log(delta)
        ptmin = jnp.minimum(pti, ptj)
        lnkt = jnp.log(ptmin * delta + EPS)
        lnz = jnp.log(ptmin / (pti + ptj + EPS) + EPS)
        m2 = ((ei + ej) ** 2 - (pxi + pxj) ** 2
              - (pyi + pyj) ** 2 - (pzi + pzj) ** 2)
        lnm2 = jnp.log(jnp.maximum(m2, EPS))
        zero4 = jnp.zeros_like(lnkt)
        edge = jnp.concatenate(
            [cF, nF - cF, lnkt, lnz, lndelta, lnm2,
             zero4, zero4, zero4, zero4], axis=1)
        h = jax.lax.dot_general(edge, w1_ref[...], dnN,
                                preferred_element_type=f32) + b1_ref[...]
        h = jnp.maximum(h, 0.0)
        msg = jax.lax.dot_general(h, w2_ref[...], dnN,
                                  preferred_element_type=f32) + b2_ref[...]
        msg = jnp.maximum(msg, 0.0)
        msg_s[k * M:(k + 1) * M, :] = msg
        att_s[:, k:k + 1] = jax.lax.dot_general(
            msg, watt_ref[...], dnN, preferred_element_type=f32)

    att = att_s[...]
    amx = jnp.max(att, axis=1, keepdims=True)
    ex = jnp.exp(att - amx)
    attw = ex / jnp.sum(ex, axis=1, keepdims=True)
    facc = jnp.zeros(out_ref.shape, dtype=f32)
    lacc = jnp.zeros(lv_ref.shape, dtype=f32)
    for k in range(K):
        w = attw[:, k:k + 1]
        facc = facc + msg_s[k * M:(k + 1) * M, :] * w
        lacc = lacc + nlv_s[k * M:(k + 1) * M, :] * w
    out_ref[...] = facc
    lv_ref[...] = lacc


def _mlp_call(nidx, featT, lvT8, cF, cLV8, noiseT8,
              w1tp, w2t, wattT, b1r, b2r):
    B, P, C = featT.shape
    Cout = w2t.shape[1]
    wspec = lambda shape: pl.BlockSpec(shape, lambda b: tuple(0 for _ in shape))
    return pl.pallas_call(
        _mlp_body,
        grid=(B,),
        in_specs=[
            pl.BlockSpec((None, K, M), lambda b: (b, 0, 0)),
            pl.BlockSpec((None, P, C), lambda b: (b, 0, 0)),
            pl.BlockSpec((None, P, 8), lambda b: (b, 0, 0)),
            pl.BlockSpec((None, M, C), lambda b: (b, 0, 0)),
            pl.BlockSpec((None, M, 8), lambda b: (b, 0, 0)),
            pl.BlockSpec((None, K, M, 8), lambda b: (b, 0, 0, 0)),
            wspec((2 * C + 8, Cout)),
            wspec((Cout, Cout)),
            wspec((Cout, 1)),
            wspec((1, Cout)),
            wspec((1, Cout)),
        ],
        out_specs=[
            pl.BlockSpec((None, M, Cout), lambda b: (b, 0, 0)),
            pl.BlockSpec((None, M, 8), lambda b: (b, 0, 0)),
        ],
        out_shape=[
            jax.ShapeDtypeStruct((B, M, Cout), jnp.float32),
            jax.ShapeDtypeStruct((B, M, 8), jnp.float32),
        ],
        scratch_shapes=[
            pltpu.VMEM((K * M, Cout), jnp.float32),
            pltpu.VMEM((K * M, 8), jnp.float32),
            pltpu.VMEM((M, K), jnp.float32),
        ],
    )(nidx, featT, lvT8, cF, cLV8, noiseT8,
      w1tp, w2t, wattT, b1r, b2r)


def kernel(features, lorentz_vectors, coordinates, mask, W1, b1, W2, b2,
           Watt, batt):
    B, Cin, P = features.shape
    f32 = jnp.float32

    eta = coordinates[:, 0, :]
    phi = coordinates[:, 1, :]
    rnd = jax.random.uniform(jax.random.key(7), (B, P))

    cidxT = _fps_call(eta.T, phi.T, rnd.T)
    cidx = cidxT.T

    featT = jnp.swapaxes(features, 1, 2)
    pad4 = jnp.zeros((B, P, 4), f32)
    lvT8 = jnp.concatenate([jnp.swapaxes(lorentz_vectors, 1, 2), pad4], axis=2)
    coordT8 = jnp.concatenate(
        [jnp.swapaxes(coordinates, 1, 2), jnp.zeros((B, P, 6), f32)], axis=2)

    nidx, cF, cLV8 = _knn_call(coordT8, cidx.reshape(B, 1, M), featT, lvT8)

    noise = jax.random.normal(jax.random.key(1), (B, 4, M, K), dtype=f32) * 1e-6
    noiseT8 = jnp.concatenate(
        [noise.transpose(0, 3, 2, 1), jnp.zeros((B, K, M, 4), f32)], axis=3)

    Cout = W2.shape[0]
    w1tp = jnp.concatenate([W1.T, jnp.zeros((4, Cout), f32)],
                           axis=0)

    out, lv8 = _mlp_call(nidx, featT, lvT8, cF, cLV8, noiseT8,
                         w1tp, W2.T, Watt.T,
                         b1[None, :], b2[None, :])
    feat_out = out.transpose(0, 2, 1)
    lv_out = lv8[:, :, :4].transpose(0, 2, 1)
    return feat_out, lv_out
